# baseline (speedup 1.0000x reference)
import jax
import jax.numpy as jnp
from jax import lax
from jax.experimental import pallas as pl
from jax.experimental.pallas import tpu as pltpu
from jax.experimental.pallas import tpu_sc as plsc

B, N, D = 64, 576, 768
R = B * N
NUM_PATCHES = 576
DW = D // 2

_info = plsc.get_sparse_core_info()
NC, NS, L = _info.num_cores, _info.num_subcores, _info.num_lanes
NW = NC * NS
ROWS_PER_W = R // NW
C = 32
NBUF = 2
NCHUNK = ROWS_PER_W // C
NROUND = NCHUNK // NBUF
MASK_HI = -65536


def _body(x_hbm, ids_hbm, pe_hbm, out_hbm,
          ids_all, xb0, xb1, pb0, pb1, sx0, sx1, sg0, sg1, so0, so1):
    xb = (xb0, xb1)
    pb = (pb0, pb1)
    sx = (sx0, sx1)
    sg = (sg0, sg1)
    so = (so0, so1)

    wid = lax.axis_index("s") * NC + lax.axis_index("c")
    base = wid * ROWS_PER_W

    pltpu.sync_copy(ids_hbm.at[pl.ds(base, ROWS_PER_W)], ids_all)

    def start(b, g, first):
        if not first:
            pltpu.make_async_copy(
                xb[b], out_hbm.at[pl.ds(base, C)], so[b]).wait()
        pltpu.async_copy(x_hbm.at[pl.ds(base + g * C, C)], xb[b], sx[b])
        pltpu.async_copy(
            pe_hbm.at[ids_all.at[pl.ds(g * C, C)]], pb[b], sg[b])

    def finish(b, g):
        pltpu.make_async_copy(
            x_hbm.at[pl.ds(base, C)], xb[b], sx[b]).wait()
        pltpu.make_async_copy(
            pe_hbm.at[ids_all.at[pl.ds(g * C, C)]], pb[b], sg[b]).wait()

        def row2(r2, rc):
            for half in range(2):
                r = r2 * 2 + half
                for gr in range(DW // L):
                    w = pb[b][r, pl.ds(gr * L, L)]
                    lo = plsc.bitcast(w << 16, jnp.float32)
                    hi = plsc.bitcast(w, jnp.float32)
                    plsc.addupdate(xb[b].at[r, pl.ds(gr * 2 * L, L)], lo)
                    plsc.addupdate(xb[b].at[r, pl.ds(gr * 2 * L + L, L)], hi)
            return rc

        lax.fori_loop(0, C // 2, row2, 0)
        pltpu.async_copy(xb[b], out_hbm.at[pl.ds(base + g * C, C)], so[b])

    start(0, 0, True)
    start(1, 1, True)
    finish(0, 0)
    start(0, 2, False)
    finish(1, 1)

    def round_(k, carry):
        for b in range(NBUF):
            g = k * NBUF + b

            @pl.when(g + 1 < NCHUNK)
            def _():
                start((b + 1) % NBUF, g + 1, False)

            finish(b, g)
        return carry

    lax.fori_loop(1, NROUND, round_, 0)

    for b in range(NBUF):
        pltpu.make_async_copy(xb[b], out_hbm.at[pl.ds(base, C)], so[b]).wait()


@jax.jit
def kernel(x, ids, pe):
    x2 = x.reshape(R, D)
    ids2 = ids.reshape(R).astype(jnp.int32)
    v = pe.reshape(NUM_PATCHES, D // 32, 2, 16).astype(jnp.bfloat16)
    s = jnp.stack([v[:, :, 0, :], v[:, :, 1, :]], axis=-1)
    pe_packed = lax.bitcast_convert_type(s, jnp.int32).reshape(
        NUM_PATCHES, DW)

    mesh = plsc.VectorSubcoreMesh(core_axis_name="c", subcore_axis_name="s")
    out = pl.kernel(
        _body,
        mesh=mesh,
        compiler_params=pltpu.CompilerParams(needs_layout_passes=False),
        out_type=jax.ShapeDtypeStruct((R, D), jnp.float32),
        scratch_types=[
            pltpu.VMEM((ROWS_PER_W,), jnp.int32),
            pltpu.VMEM((C, D), jnp.float32),
            pltpu.VMEM((C, D), jnp.float32),
            pltpu.VMEM((C, DW), jnp.int32),
            pltpu.VMEM((C, DW), jnp.int32),
        ] + [pltpu.SemaphoreType.DMA] * 6,
    )(x2, ids2, pe_packed)
    return out.reshape(1, B, N, D)


# speedup vs baseline: 1.2674x; 1.2674x over previous
import jax
import jax.numpy as jnp
from jax import lax
from jax.experimental import pallas as pl
from jax.experimental.pallas import tpu as pltpu
from jax.experimental.pallas import tpu_sc as plsc

B, N, D = 64, 576, 768
R = B * N
NUM_PATCHES = 576

_info = plsc.get_sparse_core_info()
NC, NS, L = _info.num_cores, _info.num_subcores, _info.num_lanes
NW = NC * NS
ROWS_PER_W = R // NW
C = 24
NBUF = 3
NCHUNK = ROWS_PER_W // C
NROUND = NCHUNK // NBUF


def _body(x_hbm, ids_hbm, pe_hbm, out_hbm,
          ids_all, xb0, xb1, xb2, pb0, pb1, pb2,
          sx0, sx1, sx2, sg0, sg1, sg2, so0, so1, so2):
    xb = (xb0, xb1, xb2)
    pb = (pb0, pb1, pb2)
    sx = (sx0, sx1, sx2)
    sg = (sg0, sg1, sg2)
    so = (so0, so1, so2)

    wid = lax.axis_index("s") * NC + lax.axis_index("c")
    base = wid * ROWS_PER_W

    pltpu.sync_copy(ids_hbm.at[pl.ds(base, ROWS_PER_W)], ids_all)

    def start(b, g, first):
        if not first:
            pltpu.make_async_copy(
                xb[b], out_hbm.at[pl.ds(base, C)], so[b]).wait()
        pltpu.async_copy(x_hbm.at[pl.ds(base + g * C, C)], xb[b], sx[b])
        pltpu.async_copy(
            pe_hbm.at[ids_all.at[pl.ds(g * C, C)]], pb[b], sg[b])

    def finish(b, g):
        pltpu.make_async_copy(
            x_hbm.at[pl.ds(base, C)], xb[b], sx[b]).wait()
        pltpu.make_async_copy(
            pe_hbm.at[ids_all.at[pl.ds(g * C, C)]], pb[b], sg[b]).wait()

        def row(r, rc):
            for gr in range(D // L):
                sl = pl.ds(gr * L, L)
                plsc.addupdate(xb[b].at[r, sl], pb[b][r, sl])
            return rc

        lax.fori_loop(0, C, row, 0)
        pltpu.async_copy(xb[b], out_hbm.at[pl.ds(base + g * C, C)], so[b])

    start(0, 0, True)
    start(1, 1, True)
    start(2, 2, True)

    def round_(k, carry):
        for b in range(NBUF):
            g = k * NBUF + b
            finish(b, g)

            @pl.when(g + NBUF < NCHUNK)
            def _():
                start(b, g + NBUF, False)

        return carry

    lax.fori_loop(0, NROUND, round_, 0)

    for b in range(NBUF):
        pltpu.make_async_copy(xb[b], out_hbm.at[pl.ds(base, C)], so[b]).wait()


@jax.jit
def kernel(x, ids, pe):
    x2 = x.reshape(R, D)
    ids2 = ids.reshape(R).astype(jnp.int32)
    pe2 = pe.reshape(NUM_PATCHES, D)

    mesh = plsc.VectorSubcoreMesh(core_axis_name="c", subcore_axis_name="s")
    out = pl.kernel(
        _body,
        mesh=mesh,
        out_type=jax.ShapeDtypeStruct((R, D), jnp.float32),
        scratch_types=[
            pltpu.VMEM((ROWS_PER_W,), jnp.int32),
            pltpu.VMEM((C, D), jnp.float32),
            pltpu.VMEM((C, D), jnp.float32),
            pltpu.VMEM((C, D), jnp.float32),
            pltpu.VMEM((C, D), jnp.float32),
            pltpu.VMEM((C, D), jnp.float32),
            pltpu.VMEM((C, D), jnp.float32),
        ] + [pltpu.SemaphoreType.DMA] * 9,
    )(x2, ids2, pe2)
    return out.reshape(1, B, N, D)
